# baseline (speedup 1.0000x reference)
import functools

import jax
import jax.numpy as jnp
from jax.experimental import pallas as pl
from jax.experimental.pallas import tpu as pltpu

B = 16384
D = 768
K = 4
CS = 1024
SUB = D // K
DEC_H = 512
COMMIT = 0.25

BLOCK_B = 512


PAD = 256


def _fused_kernel(x_ref, W1_ref, b1_ref, W2_ref, b2_ref, W3_ref, b3_ref,
                  W4p_ref, b4p_ref, cb_ref, cnorm_ref, Wd1_ref,
                  bd1_ref, Wd2_ref, bd2_ref, xh_ref, q_ref, codes_ref,
                  loss_ref):
    xb = x_ref[...]
    h = jnp.maximum(xb @ W1_ref[...] + b1_ref[...], 0.0)
    h = jnp.maximum(h @ W2_ref[...] + b2_ref[...], 0.0)
    h = jnp.maximum(h @ W3_ref[...] + b3_ref[...], 0.0)
    semp = h @ W4p_ref[...] + b4p_ref[...]

    qs = []
    codes = []
    pq_sum = jnp.float32(0.0)
    for k in range(K):
        t = semp[:, k * PAD:k * PAD + SUB]
        scores = jax.lax.dot_general(t, cb_ref[k], (((1,), (1,)), ((), ())))
        d = (jnp.sum(t * t, axis=1, keepdims=True)
             + cnorm_ref[k][None, :] - 2.0 * scores)
        dmin = jnp.min(d, axis=1, keepdims=True)
        iota = jax.lax.broadcasted_iota(jnp.int32, (BLOCK_B, CS), 1)
        ck = jnp.min(jnp.where(d == dmin, iota, CS), axis=1)
        onehot = (iota == ck[:, None]).astype(jnp.float32)
        qk = onehot @ cb_ref[k]
        qs.append(qk)
        codes.append(ck)
        pq_sum = pq_sum + jnp.sum(dmin)

    q = jnp.concatenate(qs, axis=1)
    hd = jnp.maximum(q @ Wd1_ref[...] + bd1_ref[...], 0.0)
    xh = hd @ Wd2_ref[...] + bd2_ref[...]

    xh_ref[...] = xh
    for k in range(K):
        q_ref[:, k, :] = qs[k]
    codes_ref[...] = jnp.stack(codes, axis=1).astype(jnp.int32)
    recon_sum = jnp.sum((xh - xb) ** 2)
    loss_ref[...] = jnp.stack([recon_sum, pq_sum]).reshape(1, 1, 2)


@functools.partial(jax.jit, static_argnums=())
def kernel(x, W1, b1, W2, b2, W3, b3, W4, b4, codebooks, Wd1, bd1, Wd2, bd2):
    nblk = B // BLOCK_B
    full = lambda shp: pl.BlockSpec(shp, lambda i: (0,) * len(shp))
    row2 = lambda n: pl.BlockSpec((BLOCK_B, n), lambda i: (i, 0))

    out_shapes = (
        jax.ShapeDtypeStruct((B, D), jnp.float32),
        jax.ShapeDtypeStruct((B, K, SUB), jnp.float32),
        jax.ShapeDtypeStruct((B, K), jnp.int32),
        jax.ShapeDtypeStruct((nblk, 1, 2), jnp.float32),
    )
    out_specs = (
        row2(D),
        pl.BlockSpec((BLOCK_B, K, SUB), lambda i: (i, 0, 0)),
        pl.BlockSpec((BLOCK_B, K), lambda i: (i, 0)),
        pl.BlockSpec((1, 1, 2), lambda i: (i, 0, 0)),
    )
    in_specs = [
        row2(D),
        full((D, 512)), full((1, 512)),
        full((512, 256)), full((1, 256)),
        full((256, 128)), full((1, 128)),
        full((128, K * PAD)), full((1, K * PAD)),
        full((K, CS, SUB)), full((K, CS)),
        full((D, DEC_H)), full((1, DEC_H)),
        full((DEC_H, D)), full((1, D)),
    ]

    wz = jnp.zeros((128, PAD - SUB), jnp.float32)
    bz = jnp.zeros((PAD - SUB,), jnp.float32)
    W4p = jnp.concatenate(
        sum([[W4[:, k * SUB:(k + 1) * SUB], wz] for k in range(K)], []),
        axis=1)
    b4p = jnp.concatenate(
        sum([[b4[k * SUB:(k + 1) * SUB], bz] for k in range(K)], []))

    cnorm = jnp.sum(codebooks * codebooks, axis=2)

    x_hat, quantized, codes, loss_parts = pl.pallas_call(
        _fused_kernel,
        grid=(nblk,),
        in_specs=in_specs,
        out_specs=out_specs,
        out_shape=out_shapes,
        compiler_params=pltpu.CompilerParams(
            dimension_semantics=("parallel",)),
    )(x, W1, b1.reshape(1, -1), W2, b2.reshape(1, -1),
      W3, b3.reshape(1, -1), W4p, b4p.reshape(1, -1), codebooks,
      cnorm, Wd1, bd1.reshape(1, -1), Wd2, bd2.reshape(1, -1))

    sums = jnp.sum(loss_parts.reshape(nblk, 2), axis=0)
    reconstruction_loss = sums[0] / (B * D)
    pqvae_loss = (1.0 + COMMIT) * sums[1] / (B * D)
    total_loss = reconstruction_loss + pqvae_loss
    return (total_loss, reconstruction_loss, pqvae_loss, codes, quantized,
            x_hat)


# speedup vs baseline: 1.2014x; 1.2014x over previous
import functools

import jax
import jax.numpy as jnp
from jax.experimental import pallas as pl
from jax.experimental.pallas import tpu as pltpu

B = 16384
D = 768
K = 4
CS = 1024
SUB = D // K
DEC_H = 512
COMMIT = 0.25

BLOCK_B = 1024


PAD = 256


def _fused_kernel(x_ref, W1_ref, b1_ref, W2_ref, b2_ref, W3_ref, b3_ref,
                  W4p_ref, b4p_ref, cb_ref, cnorm_ref, Wd1_ref,
                  bd1_ref, Wd2_ref, bd2_ref, xh_ref, q_ref, codes_ref,
                  loss_ref):
    xb = x_ref[...]
    h = jnp.maximum(xb @ W1_ref[...] + b1_ref[...], 0.0)
    h = jnp.maximum(h @ W2_ref[...] + b2_ref[...], 0.0)
    h = jnp.maximum(h @ W3_ref[...] + b3_ref[...], 0.0)
    semp = h @ W4p_ref[...] + b4p_ref[...]

    qs = []
    codes = []
    pq_sum = jnp.float32(0.0)
    for k in range(K):
        t = semp[:, k * PAD:k * PAD + SUB]
        scores = jax.lax.dot_general(t, cb_ref[k], (((1,), (1,)), ((), ())))
        d = (jnp.sum(t * t, axis=1, keepdims=True)
             + cnorm_ref[k][None, :] - 2.0 * scores)
        dmin = jnp.min(d, axis=1, keepdims=True)
        iota = jax.lax.broadcasted_iota(jnp.int32, (BLOCK_B, CS), 1)
        ck = jnp.min(jnp.where(d == dmin, iota, CS), axis=1)
        onehot = (iota == ck[:, None]).astype(jnp.float32)
        qk = onehot @ cb_ref[k]
        qs.append(qk)
        codes.append(ck)
        pq_sum = pq_sum + jnp.sum(dmin)

    q = jnp.concatenate(qs, axis=1)
    hd = jnp.maximum(q @ Wd1_ref[...] + bd1_ref[...], 0.0)
    xh = hd @ Wd2_ref[...] + bd2_ref[...]

    xh_ref[...] = xh
    q_ref[...] = q
    codes_ref[...] = jnp.stack(codes, axis=1).astype(jnp.int32)
    recon_sum = jnp.sum((xh - xb) ** 2)
    loss_ref[...] = jnp.stack([recon_sum, pq_sum]).reshape(1, 1, 2)


@functools.partial(jax.jit, static_argnums=())
def kernel(x, W1, b1, W2, b2, W3, b3, W4, b4, codebooks, Wd1, bd1, Wd2, bd2):
    nblk = B // BLOCK_B
    full = lambda shp: pl.BlockSpec(shp, lambda i: (0,) * len(shp))
    row2 = lambda n: pl.BlockSpec((BLOCK_B, n), lambda i: (i, 0))

    out_shapes = (
        jax.ShapeDtypeStruct((B, D), jnp.float32),
        jax.ShapeDtypeStruct((B, D), jnp.float32),
        jax.ShapeDtypeStruct((B, K), jnp.int32),
        jax.ShapeDtypeStruct((nblk, 1, 2), jnp.float32),
    )
    out_specs = (
        row2(D),
        row2(D),
        pl.BlockSpec((BLOCK_B, K), lambda i: (i, 0)),
        pl.BlockSpec((1, 1, 2), lambda i: (i, 0, 0)),
    )
    in_specs = [
        row2(D),
        full((D, 512)), full((1, 512)),
        full((512, 256)), full((1, 256)),
        full((256, 128)), full((1, 128)),
        full((128, K * PAD)), full((1, K * PAD)),
        full((K, CS, SUB)), full((K, CS)),
        full((D, DEC_H)), full((1, DEC_H)),
        full((DEC_H, D)), full((1, D)),
    ]

    wz = jnp.zeros((128, PAD - SUB), jnp.float32)
    bz = jnp.zeros((PAD - SUB,), jnp.float32)
    W4p = jnp.concatenate(
        sum([[W4[:, k * SUB:(k + 1) * SUB], wz] for k in range(K)], []),
        axis=1)
    b4p = jnp.concatenate(
        sum([[b4[k * SUB:(k + 1) * SUB], bz] for k in range(K)], []))

    cnorm = jnp.sum(codebooks * codebooks, axis=2)

    x_hat, q_flat, codes, loss_parts = pl.pallas_call(
        _fused_kernel,
        grid=(nblk,),
        in_specs=in_specs,
        out_specs=out_specs,
        out_shape=out_shapes,
        compiler_params=pltpu.CompilerParams(
            dimension_semantics=("parallel",)),
    )(x, W1, b1.reshape(1, -1), W2, b2.reshape(1, -1),
      W3, b3.reshape(1, -1), W4p, b4p.reshape(1, -1), codebooks,
      cnorm, Wd1, bd1.reshape(1, -1), Wd2, bd2.reshape(1, -1))

    sums = jnp.sum(loss_parts.reshape(nblk, 2), axis=0)
    reconstruction_loss = sums[0] / (B * D)
    pqvae_loss = (1.0 + COMMIT) * sums[1] / (B * D)
    total_loss = reconstruction_loss + pqvae_loss
    quantized = q_flat.reshape(B, K, SUB)
    return (total_loss, reconstruction_loss, pqvae_loss, codes, quantized,
            x_hat)


# speedup vs baseline: 1.2442x; 1.0357x over previous
import functools

import jax
import jax.numpy as jnp
from jax.experimental import pallas as pl
from jax.experimental.pallas import tpu as pltpu

B = 16384
D = 768
K = 4
CS = 1024
SUB = D // K
DEC_H = 512
COMMIT = 0.25

BLOCK_B = 1024


PAD = 256


def _fused_kernel(x_ref, W1_ref, b1_ref, W2_ref, b2_ref, W3_ref, b3_ref,
                  W4p_ref, b4p_ref, cb_ref, cnorm_ref, Wd1_ref,
                  bd1_ref, Wd2_ref, bd2_ref, xh_ref, q_ref, codes_ref,
                  loss_ref):
    xb = x_ref[...]
    h = jnp.maximum(xb @ W1_ref[...] + b1_ref[...], 0.0)
    h = jnp.maximum(h @ W2_ref[...] + b2_ref[...], 0.0)
    h = jnp.maximum(h @ W3_ref[...] + b3_ref[...], 0.0)
    semp = h @ W4p_ref[...] + b4p_ref[...]

    qs = []
    codes = []
    pq_sum = jnp.float32(0.0)
    for k in range(K):
        t = semp[:, k * PAD:k * PAD + SUB]
        scores = jax.lax.dot_general(t, cb_ref[k], (((1,), (1,)), ((), ())))
        d = cnorm_ref[k][None, :] - 2.0 * scores
        dmin = jnp.min(d, axis=1, keepdims=True)
        iota = jax.lax.broadcasted_iota(jnp.int32, (BLOCK_B, CS), 1)
        ck = jnp.min(jnp.where(d == dmin, iota, CS), axis=1)
        onehot = (iota == ck[:, None]).astype(jnp.float32)
        qk = onehot @ cb_ref[k]
        qs.append(qk)
        codes.append(ck)
        pq_sum = pq_sum + (jnp.sum(t * t) + jnp.sum(dmin))

    q = jnp.concatenate(qs, axis=1)
    hd = jnp.maximum(q @ Wd1_ref[...] + bd1_ref[...], 0.0)
    xh = hd @ Wd2_ref[...] + bd2_ref[...]

    xh_ref[...] = xh
    q_ref[...] = q
    codes_ref[...] = jnp.stack(codes, axis=1).astype(jnp.int32)
    recon_sum = jnp.sum((xh - xb) ** 2)
    loss_ref[...] = jnp.stack([recon_sum, pq_sum]).reshape(1, 1, 2)


@functools.partial(jax.jit, static_argnums=())
def kernel(x, W1, b1, W2, b2, W3, b3, W4, b4, codebooks, Wd1, bd1, Wd2, bd2):
    nblk = B // BLOCK_B
    full = lambda shp: pl.BlockSpec(shp, lambda i: (0,) * len(shp))
    row2 = lambda n: pl.BlockSpec((BLOCK_B, n), lambda i: (i, 0))

    out_shapes = (
        jax.ShapeDtypeStruct((B, D), jnp.float32),
        jax.ShapeDtypeStruct((B, D), jnp.float32),
        jax.ShapeDtypeStruct((B, K), jnp.int32),
        jax.ShapeDtypeStruct((nblk, 1, 2), jnp.float32),
    )
    out_specs = (
        row2(D),
        row2(D),
        pl.BlockSpec((BLOCK_B, K), lambda i: (i, 0)),
        pl.BlockSpec((1, 1, 2), lambda i: (i, 0, 0)),
    )
    in_specs = [
        row2(D),
        full((D, 512)), full((1, 512)),
        full((512, 256)), full((1, 256)),
        full((256, 128)), full((1, 128)),
        full((128, K * PAD)), full((1, K * PAD)),
        full((K, CS, SUB)), full((K, CS)),
        full((D, DEC_H)), full((1, DEC_H)),
        full((DEC_H, D)), full((1, D)),
    ]

    wz = jnp.zeros((128, PAD - SUB), jnp.float32)
    bz = jnp.zeros((PAD - SUB,), jnp.float32)
    W4p = jnp.concatenate(
        sum([[W4[:, k * SUB:(k + 1) * SUB], wz] for k in range(K)], []),
        axis=1)
    b4p = jnp.concatenate(
        sum([[b4[k * SUB:(k + 1) * SUB], bz] for k in range(K)], []))

    cnorm = jnp.sum(codebooks * codebooks, axis=2)

    x_hat, q_flat, codes, loss_parts = pl.pallas_call(
        _fused_kernel,
        grid=(nblk,),
        in_specs=in_specs,
        out_specs=out_specs,
        out_shape=out_shapes,
        compiler_params=pltpu.CompilerParams(
            dimension_semantics=("parallel",)),
    )(x, W1, b1.reshape(1, -1), W2, b2.reshape(1, -1),
      W3, b3.reshape(1, -1), W4p, b4p.reshape(1, -1), codebooks,
      cnorm, Wd1, bd1.reshape(1, -1), Wd2, bd2.reshape(1, -1))

    sums = jnp.sum(loss_parts.reshape(nblk, 2), axis=0)
    reconstruction_loss = sums[0] / (B * D)
    pqvae_loss = (1.0 + COMMIT) * sums[1] / (B * D)
    total_loss = reconstruction_loss + pqvae_loss
    quantized = q_flat.reshape(B, K, SUB)
    return (total_loss, reconstruction_loss, pqvae_loss, codes, quantized,
            x_hat)
